# Initial kernel scaffold; baseline (speedup 1.0000x reference)
import functools

import jax
import jax.numpy as jnp
from jax import lax
from jax.experimental import pallas as pl
from jax.experimental.pallas import tpu as pltpu
from jax.experimental.pallas import tpu_sc as plsc

VOCAB = 5053
EMBED_DIM = 32

NUM_CORES = 2
NUM_SUBCORES = 16
NUM_WORKERS = NUM_CORES * NUM_SUBCORES

CHUNK = 1024


@jax.jit
def _encode(x_flat, table):
    n = x_flat.shape[0]
    per_worker = n // NUM_WORKERS
    n_chunks = per_worker // CHUNK

    mesh = plsc.VectorSubcoreMesh(
        core_axis_name="c", subcore_axis_name="s",
        num_cores=NUM_CORES, num_subcores=NUM_SUBCORES,
    )

    @functools.partial(
        pl.kernel,
        out_type=jax.ShapeDtypeStruct((n, EMBED_DIM), jnp.float32),
        mesh=mesh,
        scratch_types=[
            pltpu.VMEM((CHUNK,), jnp.int32),
            pltpu.VMEM((CHUNK, EMBED_DIM), jnp.float32),
            pltpu.SemaphoreType.DMA,
        ],
    )
    def gather_kernel(x_hbm, table_hbm, out_hbm, idx_v, rows_v, sem):
        wid = lax.axis_index("s") * NUM_CORES + lax.axis_index("c")
        base = wid * per_worker

        def chunk_body(i, carry):
            off = base + i * CHUNK
            pltpu.sync_copy(x_hbm.at[pl.ds(off, CHUNK)], idx_v)
            pltpu.async_copy(table_hbm.at[idx_v], rows_v, sem).wait()
            pltpu.sync_copy(rows_v, out_hbm.at[pl.ds(off, CHUNK), :])
            return carry

        lax.fori_loop(0, n_chunks, chunk_body, 0)

    return gather_kernel(x_flat, table)


def kernel(x, table):
    b, s = x.shape
    x_flat = jnp.reshape(x, (b * s,)).astype(jnp.int32)
    out = _encode(x_flat, table)
    return jnp.reshape(out, (b, s, EMBED_DIM))


# speedup vs baseline: 1.9624x; 1.9624x over previous
import functools

import jax
import jax.numpy as jnp
from jax import lax
from jax.experimental import pallas as pl
from jax.experimental.pallas import tpu as pltpu
from jax.experimental.pallas import tpu_sc as plsc

VOCAB = 5053
EMBED_DIM = 32

NUM_CORES = 2
NUM_SUBCORES = 16
NUM_WORKERS = NUM_CORES * NUM_SUBCORES

CHUNK = 1024


@jax.jit
def _encode(x_flat, table):
    n = x_flat.shape[0]
    per_worker = n // NUM_WORKERS
    n_chunks = per_worker // CHUNK

    mesh = plsc.VectorSubcoreMesh(
        core_axis_name="c", subcore_axis_name="s",
        num_cores=NUM_CORES, num_subcores=NUM_SUBCORES,
    )

    @functools.partial(
        pl.kernel,
        out_type=jax.ShapeDtypeStruct((n, EMBED_DIM), jnp.float32),
        mesh=mesh,
        scratch_types=[
            pltpu.VMEM((CHUNK,), jnp.int32),
            pltpu.VMEM((CHUNK, EMBED_DIM), jnp.float32),
            pltpu.SemaphoreType.DMA,
        ],
        compiler_params=pltpu.CompilerParams(use_tc_tiling_on_sc=False),
    )
    def gather_kernel(x_hbm, table_hbm, out_hbm, idx_v, rows_v, sem):
        wid = lax.axis_index("s") * NUM_CORES + lax.axis_index("c")
        base = wid * per_worker

        def chunk_body(i, carry):
            off = base + i * CHUNK
            pltpu.sync_copy(x_hbm.at[pl.ds(off, CHUNK)], idx_v)
            pltpu.async_copy(table_hbm.at[idx_v], rows_v, sem).wait()
            pltpu.sync_copy(rows_v, out_hbm.at[pl.ds(off, CHUNK), :])
            return carry

        lax.fori_loop(0, n_chunks, chunk_body, 0)

    return gather_kernel(x_flat, table)


def kernel(x, table):
    b, s = x.shape
    x_flat = jnp.reshape(x, (b * s,)).astype(jnp.int32)
    out = _encode(x_flat, table)
    return jnp.reshape(out, (b, s, EMBED_DIM))


# speedup vs baseline: 1.9734x; 1.0056x over previous
import functools

import jax
import jax.numpy as jnp
from jax import lax
from jax.experimental import pallas as pl
from jax.experimental.pallas import tpu as pltpu
from jax.experimental.pallas import tpu_sc as plsc

VOCAB = 5053
EMBED_DIM = 32

NUM_CORES = 2
NUM_SUBCORES = 16
NUM_WORKERS = NUM_CORES * NUM_SUBCORES

CHUNK = 1024
NBUF = 2


@jax.jit
def _encode(x_flat, table):
    n = x_flat.shape[0]
    per_worker = n // NUM_WORKERS
    n_chunks = per_worker // CHUNK
    n_outer = n_chunks // NBUF

    mesh = plsc.VectorSubcoreMesh(
        core_axis_name="c", subcore_axis_name="s",
        num_cores=NUM_CORES, num_subcores=NUM_SUBCORES,
    )

    @functools.partial(
        pl.kernel,
        out_type=jax.ShapeDtypeStruct((n, EMBED_DIM), jnp.float32),
        mesh=mesh,
        scratch_types=[
            pltpu.VMEM((NBUF, CHUNK), jnp.int32),
            pltpu.VMEM((NBUF, CHUNK, EMBED_DIM), jnp.float32),
            pltpu.VMEM_SHARED((VOCAB, EMBED_DIM), jnp.float32),
            pltpu.SemaphoreType.DMA((NBUF,)),
            pltpu.SemaphoreType.DMA((NBUF,)),
            pltpu.SemaphoreType.DMA((NBUF,)),
            pltpu.SemaphoreType.DMA,
        ],
        compiler_params=pltpu.CompilerParams(use_tc_tiling_on_sc=False),
    )
    def gather_kernel(x_hbm, table_hbm, out_hbm,
                      idx_v, rows_v, tab_sh, sem_i, sem_g, sem_o, sem_t):
        cid = lax.axis_index("c")
        sid = lax.axis_index("s")
        wid = sid * NUM_CORES + cid
        base = wid * per_worker

        @pl.when(sid == 0)
        def _():
            pltpu.async_copy(table_hbm, tab_sh, sem_t).wait()

        plsc.subcore_barrier()

        for b in range(NBUF):
            pltpu.async_copy(
                x_hbm.at[pl.ds(base + b * CHUNK, CHUNK)], idx_v.at[b],
                sem_i.at[b])

        def outer(g, carry):
            for b in range(NBUF):
                off = base + (g * NBUF + b) * CHUNK
                pltpu.make_async_copy(
                    x_hbm.at[pl.ds(off, CHUNK)], idx_v.at[b],
                    sem_i.at[b]).wait()

                @pl.when(g > 0)
                def _():
                    prev = off - NBUF * CHUNK
                    pltpu.make_async_copy(
                        rows_v.at[b], out_hbm.at[pl.ds(prev, CHUNK), :],
                        sem_o.at[b]).wait()

                gather = pltpu.async_copy(
                    table_hbm.at[idx_v.at[b]], rows_v.at[b], sem_g.at[b])

                gather.wait()

                @pl.when(g < n_outer - 1)
                def _():
                    nxt = off + NBUF * CHUNK
                    pltpu.async_copy(
                        x_hbm.at[pl.ds(nxt, CHUNK)], idx_v.at[b],
                        sem_i.at[b])

                pltpu.async_copy(
                    rows_v.at[b], out_hbm.at[pl.ds(off, CHUNK), :],
                    sem_o.at[b])
            return carry

        lax.fori_loop(0, n_outer, outer, 0)

        for b in range(NBUF):
            last = base + (n_chunks - NBUF + b) * CHUNK
            pltpu.make_async_copy(
                rows_v.at[b], out_hbm.at[pl.ds(last, CHUNK), :],
                sem_o.at[b]).wait()

    return gather_kernel(x_flat, table)


def kernel(x, table):
    b, s = x.shape
    x_flat = jnp.reshape(x, (b * s,)).astype(jnp.int32)
    out = _encode(x_flat, table)
    return jnp.reshape(out, (b, s, EMBED_DIM))


# speedup vs baseline: 2.0153x; 1.0212x over previous
import functools

import jax
import jax.numpy as jnp
from jax import lax
from jax.experimental import pallas as pl
from jax.experimental.pallas import tpu as pltpu
from jax.experimental.pallas import tpu_sc as plsc

VOCAB = 5053
EMBED_DIM = 32

NUM_CORES = 2
NUM_SUBCORES = 16
NUM_WORKERS = NUM_CORES * NUM_SUBCORES

CHUNK = 1024
NBUF = 2


@jax.jit
def _encode(x_flat, table):
    n = x_flat.shape[0]
    per_worker = n // NUM_WORKERS
    n_chunks = per_worker // CHUNK
    n_outer = n_chunks // NBUF

    mesh = plsc.VectorSubcoreMesh(
        core_axis_name="c", subcore_axis_name="s",
        num_cores=NUM_CORES, num_subcores=NUM_SUBCORES,
    )

    @functools.partial(
        pl.kernel,
        out_type=jax.ShapeDtypeStruct((n, EMBED_DIM), jnp.float32),
        mesh=mesh,
        scratch_types=[
            pltpu.VMEM((NBUF, CHUNK), jnp.int32),
            pltpu.VMEM((NBUF, CHUNK, EMBED_DIM), jnp.float32),
            pltpu.VMEM_SHARED((VOCAB, EMBED_DIM), jnp.float32),
            pltpu.SemaphoreType.DMA((NBUF,)),
            pltpu.SemaphoreType.DMA((NBUF,)),
            pltpu.SemaphoreType.DMA((NBUF,)),
            pltpu.SemaphoreType.DMA,
        ],
        compiler_params=pltpu.CompilerParams(use_tc_tiling_on_sc=False),
    )
    def gather_kernel(x_hbm, table_hbm, out_hbm,
                      idx_v, rows_v, tab_sh, sem_i, sem_g, sem_o, sem_t):
        cid = lax.axis_index("c")
        sid = lax.axis_index("s")
        wid = sid * NUM_CORES + cid
        base = wid * per_worker

        @pl.when(sid == 0)
        def _():
            pltpu.async_copy(table_hbm, tab_sh, sem_t).wait()

        plsc.subcore_barrier()

        for b in range(NBUF):
            pltpu.async_copy(
                x_hbm.at[pl.ds(base + b * CHUNK, CHUNK)], idx_v.at[b],
                sem_i.at[b])

        def outer(g, carry):
            for b in range(NBUF):
                off = base + (g * NBUF + b) * CHUNK
                pltpu.make_async_copy(
                    x_hbm.at[pl.ds(off, CHUNK)], idx_v.at[b],
                    sem_i.at[b]).wait()

                @pl.when(g > 0)
                def _():
                    prev = off - NBUF * CHUNK
                    pltpu.make_async_copy(
                        rows_v.at[b], out_hbm.at[pl.ds(prev, CHUNK), :],
                        sem_o.at[b]).wait()

                gather = pltpu.async_copy(
                    tab_sh.at[idx_v.at[b]], rows_v.at[b], sem_g.at[b])

                gather.wait()

                @pl.when(g < n_outer - 1)
                def _():
                    nxt = off + NBUF * CHUNK
                    pltpu.async_copy(
                        x_hbm.at[pl.ds(nxt, CHUNK)], idx_v.at[b],
                        sem_i.at[b])

                pltpu.async_copy(
                    rows_v.at[b], out_hbm.at[pl.ds(off, CHUNK), :],
                    sem_o.at[b])
            return carry

        lax.fori_loop(0, n_outer, outer, 0)

        for b in range(NBUF):
            last = base + (n_chunks - NBUF + b) * CHUNK
            pltpu.make_async_copy(
                rows_v.at[b], out_hbm.at[pl.ds(last, CHUNK), :],
                sem_o.at[b]).wait()

    return gather_kernel(x_flat, table)


def kernel(x, table):
    b, s = x.shape
    x_flat = jnp.reshape(x, (b * s,)).astype(jnp.int32)
    out = _encode(x_flat, table)
    return jnp.reshape(out, (b, s, EMBED_DIM))


# speedup vs baseline: 22.3852x; 11.1076x over previous
import functools

import jax
import jax.numpy as jnp
from jax import lax
from jax.experimental import pallas as pl
from jax.experimental.pallas import tpu as pltpu
from jax.experimental.pallas import tpu_sc as plsc

VOCAB = 5053
EMBED_DIM = 32
SEQ = 100
BATCH = 16384

NUM_CORES = 2
NUM_SUBCORES = 16
NUM_WORKERS = NUM_CORES * NUM_SUBCORES

HALF_E = EMBED_DIM // 2
TC_PER_WORKER = (BATCH // 128) // (NUM_WORKERS // 2)
S_TILES_FULL = SEQ // 8
S_REM = SEQ - S_TILES_FULL * 8
UNITS_A = S_TILES_FULL * TC_PER_WORKER


def _encode(x_p, table_p):
    mesh = plsc.VectorSubcoreMesh(
        core_axis_name="c", subcore_axis_name="s",
        num_cores=NUM_CORES, num_subcores=NUM_SUBCORES,
    )

    @functools.partial(
        pl.kernel,
        out_type=jax.ShapeDtypeStruct((SEQ, EMBED_DIM, BATCH), jnp.float32),
        mesh=mesh,
        scratch_types=[
            pltpu.VMEM((HALF_E, VOCAB), jnp.float32),
            pltpu.VMEM((2, 8, 128), jnp.int32),
            pltpu.VMEM((8, HALF_E, 128), jnp.float32),
            pltpu.SemaphoreType.DMA,
            pltpu.SemaphoreType.DMA((2,)),
            pltpu.SemaphoreType.DMA((8,)),
        ],
        compiler_params=pltpu.CompilerParams(needs_layout_passes=False),
    )
    def gather_kernel(x_hbm, tab_hbm, out_hbm,
                      tab_v, xbuf, obuf, sem_t, sem_x, sem_o):
        cid = lax.axis_index("c")
        sid = lax.axis_index("s")
        wid = sid * NUM_CORES + cid
        h = wid // (NUM_WORKERS // 2)
        grp = wid % (NUM_WORKERS // 2)
        e_base = h * HALF_E
        tc_base = grp * TC_PER_WORKER

        pltpu.async_copy(
            tab_hbm.at[pl.ds(e_base, HALF_E), :], tab_v, sem_t).wait()

        def unit_coords(t):
            s_t = t // TC_PER_WORKER
            tc = tc_base + lax.rem(t, TC_PER_WORKER)
            return s_t * 8, tc * 128

        def start_xload(t, p):
            s0, b0 = unit_coords(t)
            pltpu.async_copy(
                x_hbm.at[pl.ds(s0, 8), pl.ds(b0, 128)], xbuf.at[p],
                sem_x.at[p])

        def wait_xload(t, p):
            s0, b0 = unit_coords(t)
            pltpu.make_async_copy(
                x_hbm.at[pl.ds(s0, 8), pl.ds(b0, 128)], xbuf.at[p],
                sem_x.at[p]).wait()

        def wait_oblock(s_r):
            pltpu.make_async_copy(
                obuf.at[s_r],
                out_hbm.at[0, pl.ds(e_base, HALF_E), pl.ds(0, 128)],
                sem_o.at[s_r]).wait()

        def compute_srow(p, s_r):
            vs = [xbuf[p, s_r, pl.ds(g * 16, 16)] for g in range(8)]

            def erow(e_r, carry):
                e_splat = jnp.full((16,), e_r, jnp.int32)
                for g in range(8):
                    vals = plsc.load_gather(tab_v, [e_splat, vs[g]])
                    obuf[s_r, e_r, pl.ds(g * 16, 16)] = vals
                return carry

            lax.fori_loop(0, HALF_E, erow, 0)

        def store_srow(t, s_r):
            s0, b0 = unit_coords(t)
            pltpu.async_copy(
                obuf.at[s_r],
                out_hbm.at[s0 + s_r, pl.ds(e_base, HALF_E), pl.ds(b0, 128)],
                sem_o.at[s_r])

        start_xload(0, 0)

        def outer(tp, carry):
            for p in range(2):
                t = tp * 2 + p
                wait_xload(t, p)

                @pl.when(t < UNITS_A - 1)
                def _():
                    start_xload(t + 1, 1 - p)

                for s_r in range(8):
                    @pl.when(t >= 1)
                    def _():
                        wait_oblock(s_r)
                    compute_srow(p, s_r)
                    store_srow(t, s_r)
            return carry

        lax.fori_loop(0, UNITS_A // 2, outer, 0)

        for j in range(TC_PER_WORKER):
            b0 = (tc_base + j) * 128
            pltpu.async_copy(
                x_hbm.at[pl.ds(S_TILES_FULL * 8, S_REM), pl.ds(b0, 128)],
                xbuf.at[0, pl.ds(0, S_REM)], sem_x.at[0]).wait()
            for s_r in range(S_REM):
                wait_oblock(s_r)
                compute_srow(0, s_r)
                pltpu.async_copy(
                    obuf.at[s_r],
                    out_hbm.at[S_TILES_FULL * 8 + s_r,
                               pl.ds(e_base, HALF_E), pl.ds(b0, 128)],
                    sem_o.at[s_r])

        for s_r in range(S_REM):
            wait_oblock(s_r)
        for s_r in range(S_REM, 8):
            wait_oblock(s_r)

    return gather_kernel(x_p, table_p)


def kernel(x, table):
    x_p = jnp.transpose(x.astype(jnp.int32), (1, 0))
    tab_p = jnp.transpose(table, (1, 0))
    out_p = _encode(x_p, tab_p)
    return jnp.transpose(out_p, (2, 0, 1))


# speedup vs baseline: 49.6707x; 2.2189x over previous
import functools

import jax
import jax.numpy as jnp
from jax import lax
from jax.experimental import pallas as pl
from jax.experimental.pallas import tpu as pltpu
from jax.experimental.pallas import tpu_sc as plsc

VOCAB = 5053
EMBED_DIM = 32
SEQ = 100
BATCH = 16384

NUM_CORES = 2
NUM_SUBCORES = 16
NUM_WORKERS = NUM_CORES * NUM_SUBCORES

HALF_E = EMBED_DIM // 2
TC_PER_WORKER = (BATCH // 128) // (NUM_WORKERS // 2)
S_TILES_FULL = SEQ // 8
S_REM = SEQ - S_TILES_FULL * 8
UNITS_A = S_TILES_FULL * TC_PER_WORKER
VSTRIDE = 5056


def _encode(x_p, table_p):
    mesh = plsc.VectorSubcoreMesh(
        core_axis_name="c", subcore_axis_name="s",
        num_cores=NUM_CORES, num_subcores=NUM_SUBCORES,
    )

    @functools.partial(
        pl.kernel,
        out_type=jax.ShapeDtypeStruct((SEQ, EMBED_DIM, BATCH), jnp.float32),
        mesh=mesh,
        scratch_types=[
            pltpu.VMEM((HALF_E, VOCAB), jnp.float32),
            pltpu.VMEM((2, 8, 128), jnp.int32),
            pltpu.VMEM((8, HALF_E, 128), jnp.float32),
            pltpu.SemaphoreType.DMA,
            pltpu.SemaphoreType.DMA((2,)),
            pltpu.SemaphoreType.DMA((8,)),
        ],
        compiler_params=pltpu.CompilerParams(needs_layout_passes=False),
    )
    def gather_kernel(x_hbm, tab_hbm, out_hbm,
                      tab_v, xbuf, obuf, sem_t, sem_x, sem_o):
        cid = lax.axis_index("c")
        sid = lax.axis_index("s")
        wid = sid * NUM_CORES + cid
        h = wid // (NUM_WORKERS // 2)
        grp = wid % (NUM_WORKERS // 2)
        e_base = h * HALF_E
        tc_base = grp * TC_PER_WORKER

        pltpu.async_copy(
            tab_hbm.at[pl.ds(e_base, HALF_E), :], tab_v, sem_t).wait()

        def unit_coords(t):
            s_t = t // TC_PER_WORKER
            tc = tc_base + lax.rem(t, TC_PER_WORKER)
            return s_t * 8, tc * 128

        def start_xload(t, p):
            s0, b0 = unit_coords(t)
            pltpu.async_copy(
                x_hbm.at[pl.ds(s0, 8), pl.ds(b0, 128)], xbuf.at[p],
                sem_x.at[p])

        def wait_xload(t, p):
            s0, b0 = unit_coords(t)
            pltpu.make_async_copy(
                x_hbm.at[pl.ds(s0, 8), pl.ds(b0, 128)], xbuf.at[p],
                sem_x.at[p]).wait()

        def wait_oblock(s_r):
            pltpu.make_async_copy(
                obuf.at[s_r],
                out_hbm.at[0, pl.ds(e_base, HALF_E), pl.ds(0, 128)],
                sem_o.at[s_r]).wait()

        def compute_srow(p, s_r):
            vs = [xbuf[p, s_r, pl.ds(g * 16, 16)] for g in range(8)]

            @plsc.parallel_loop(0, HALF_E)
            def erow(e_r):
                e_splat = jnp.full((16,), e_r, jnp.int32)
                for g in range(8):
                    vals = plsc.load_gather(tab_v, [e_splat, vs[g]])
                    obuf[s_r, e_r, pl.ds(g * 16, 16)] = vals

        def store_srow(t, s_r):
            s0, b0 = unit_coords(t)
            pltpu.async_copy(
                obuf.at[s_r],
                out_hbm.at[s0 + s_r, pl.ds(e_base, HALF_E), pl.ds(b0, 128)],
                sem_o.at[s_r])

        start_xload(0, 0)

        def outer(tp, carry):
            for p in range(2):
                t = tp * 2 + p
                wait_xload(t, p)

                @pl.when(t < UNITS_A - 1)
                def _():
                    start_xload(t + 1, 1 - p)

                for s_r in range(8):
                    @pl.when(t >= 1)
                    def _():
                        wait_oblock(s_r)
                    compute_srow(p, s_r)
                    store_srow(t, s_r)
            return carry

        lax.fori_loop(0, UNITS_A // 2, outer, 0)

        for j in range(TC_PER_WORKER):
            b0 = (tc_base + j) * 128
            pltpu.async_copy(
                x_hbm.at[pl.ds(S_TILES_FULL * 8, S_REM), pl.ds(b0, 128)],
                xbuf.at[0, pl.ds(0, S_REM)], sem_x.at[0]).wait()
            for s_r in range(S_REM):
                wait_oblock(s_r)
                compute_srow(0, s_r)
                pltpu.async_copy(
                    obuf.at[s_r],
                    out_hbm.at[S_TILES_FULL * 8 + s_r,
                               pl.ds(e_base, HALF_E), pl.ds(b0, 128)],
                    sem_o.at[s_r])

        for s_r in range(S_REM):
            wait_oblock(s_r)
        for s_r in range(S_REM, 8):
            wait_oblock(s_r)

    return gather_kernel(x_p, table_p)


def kernel(x, table):
    x_p = jnp.transpose(x.astype(jnp.int32), (1, 0))
    tab_p = jnp.transpose(table, (1, 0))
    out_p = _encode(x_p, tab_p)
    return jnp.transpose(out_p, (2, 0, 1))
